# baseline (speedup 1.0000x reference)
import jax
import jax.numpy as jnp
from jax.experimental import pallas as pl
from jax.experimental.pallas import tpu as pltpu
from jax.experimental.pallas import tpu_sc as plsc

_ROWS = 128
_ROW_LEN = 160000
_BS = _ROWS // 2
_NUM_WORKERS = 32
_RPW = _ROWS // _NUM_WORKERS
_NBUF = 5
_CHUNK = 16000
_CPR = _ROW_LEN // _CHUNK
_ITEMS = _RPW * _CPR

_PERM = (22, 18, 6, 26, 21, 45, 60, 39, 61, 49, 38, 27, 32, 57, 10, 63,
         35, 20, 24, 56, 52, 40, 51, 42, 55, 4, 31, 14, 0, 43, 34, 3,
         50, 5, 17, 37, 28, 2, 41, 23, 58, 44, 54, 48, 46, 36, 1, 8,
         16, 33, 30, 7, 19, 15, 9, 62, 13, 11, 59, 47, 25, 53, 12, 29)
_SRC_ROWS = _PERM + tuple(range(_BS, _ROWS))


def _sc_permute_copy(src2d):
    mesh = plsc.VectorSubcoreMesh(core_axis_name="c", subcore_axis_name="s")

    @pl.kernel(
        out_type=jax.ShapeDtypeStruct((_ROWS, _ROW_LEN), jnp.float32),
        mesh=mesh,
        compiler_params=pltpu.CompilerParams(use_tc_tiling_on_sc=False),
        scratch_types=[
            pltpu.VMEM((_NBUF, _CHUNK), jnp.float32),
            pltpu.SMEM((_RPW,), jnp.int32),
        ]
        + [pltpu.SemaphoreType.DMA] * (2 * _NBUF),
    )
    def k(src_hbm, out_hbm, bufs, srows, *sems):
        in_sems = sems[:_NBUF]
        out_sems = sems[_NBUF:]
        wid = jax.lax.axis_index("s") * 2 + jax.lax.axis_index("c")
        dst_base = wid * _RPW

        for w in range(_NUM_WORKERS):

            @pl.when(wid == w)
            def _(w=w):
                for i in range(_RPW):
                    srows[i] = _SRC_ROWS[w * _RPW + i]

        def in_copy(b, it):
            r = jax.lax.div(it, _CPR)
            off = jax.lax.mul(jax.lax.rem(it, _CPR), _CHUNK)
            return pltpu.make_async_copy(
                src_hbm.at[srows[r], pl.ds(off, _CHUNK)],
                bufs.at[b],
                in_sems[b],
            )

        def out_copy(b, it):
            r = jax.lax.div(it, _CPR)
            off = jax.lax.mul(jax.lax.rem(it, _CPR), _CHUNK)
            return pltpu.make_async_copy(
                bufs.at[b],
                out_hbm.at[dst_base + r, pl.ds(off, _CHUNK)],
                out_sems[b],
            )

        for b in range(_NBUF):
            in_copy(b, jnp.int32(b)).start()

        @pl.loop(0, _ITEMS, step=_NBUF)
        def _(t):
            for b in range(_NBUF):
                it = t + b
                in_copy(b, it).wait()
                out_copy(b, it).start()
            for b in range(_NBUF):
                it = t + b
                nxt = it + _NBUF

                @pl.when(nxt < _ITEMS)
                def _(b=b, it=it, nxt=nxt):
                    out_copy(b, it).wait()
                    in_copy(b, nxt).start()

        for b in range(_NBUF):
            out_copy(b, jnp.int32(_ITEMS - _NBUF + b)).wait()

    return k(src2d)


def kernel(sources):
    src2d = sources.reshape(_ROWS, _ROW_LEN)
    out = _sc_permute_copy(src2d)
    return out.reshape(2, _BS, 1, _ROW_LEN)


# speedup vs baseline: 1.0222x; 1.0222x over previous
import jax
import jax.numpy as jnp
from jax.experimental import pallas as pl
from jax.experimental.pallas import tpu as pltpu
from jax.experimental.pallas import tpu_sc as plsc

_ROWS = 128
_ROW_LEN = 160000
_BS = _ROWS // 2
_NUM_WORKERS = 32
_RPW = _ROWS // _NUM_WORKERS
_NBUF = 10
_CHUNK = 8000
_CPR = _ROW_LEN // _CHUNK
_ITEMS = _RPW * _CPR

_PERM = (22, 18, 6, 26, 21, 45, 60, 39, 61, 49, 38, 27, 32, 57, 10, 63,
         35, 20, 24, 56, 52, 40, 51, 42, 55, 4, 31, 14, 0, 43, 34, 3,
         50, 5, 17, 37, 28, 2, 41, 23, 58, 44, 54, 48, 46, 36, 1, 8,
         16, 33, 30, 7, 19, 15, 9, 62, 13, 11, 59, 47, 25, 53, 12, 29)
_SRC_ROWS = _PERM + tuple(range(_BS, _ROWS))


def _sc_permute_copy(src2d):
    mesh = plsc.VectorSubcoreMesh(core_axis_name="c", subcore_axis_name="s")

    @pl.kernel(
        out_type=jax.ShapeDtypeStruct((_ROWS, _ROW_LEN), jnp.float32),
        mesh=mesh,
        compiler_params=pltpu.CompilerParams(use_tc_tiling_on_sc=False),
        scratch_types=[
            pltpu.VMEM((_NBUF, _CHUNK), jnp.float32),
            pltpu.SMEM((_RPW,), jnp.int32),
        ]
        + [pltpu.SemaphoreType.DMA] * (2 * _NBUF),
    )
    def k(src_hbm, out_hbm, bufs, srows, *sems):
        in_sems = sems[:_NBUF]
        out_sems = sems[_NBUF:]
        wid = jax.lax.axis_index("s") * 2 + jax.lax.axis_index("c")
        dst_base = wid * _RPW

        for w in range(_NUM_WORKERS):

            @pl.when(wid == w)
            def _(w=w):
                for i in range(_RPW):
                    srows[i] = _SRC_ROWS[w * _RPW + i]

        def in_copy(b, it):
            r = jax.lax.div(it, _CPR)
            off = jax.lax.mul(jax.lax.rem(it, _CPR), _CHUNK)
            return pltpu.make_async_copy(
                src_hbm.at[srows[r], pl.ds(off, _CHUNK)],
                bufs.at[b],
                in_sems[b],
            )

        def out_copy(b, it):
            r = jax.lax.div(it, _CPR)
            off = jax.lax.mul(jax.lax.rem(it, _CPR), _CHUNK)
            return pltpu.make_async_copy(
                bufs.at[b],
                out_hbm.at[dst_base + r, pl.ds(off, _CHUNK)],
                out_sems[b],
            )

        for b in range(_NBUF):
            in_copy(b, jnp.int32(b)).start()

        @pl.loop(0, _ITEMS, step=_NBUF)
        def _(t):
            for b in range(_NBUF):
                it = t + b
                in_copy(b, it).wait()
                out_copy(b, it).start()
            for b in range(_NBUF):
                it = t + b
                nxt = it + _NBUF

                @pl.when(nxt < _ITEMS)
                def _(b=b, it=it, nxt=nxt):
                    out_copy(b, it).wait()
                    in_copy(b, nxt).start()

        for b in range(_NBUF):
            out_copy(b, jnp.int32(_ITEMS - _NBUF + b)).wait()

    return k(src2d)


def kernel(sources):
    src2d = sources.reshape(_ROWS, _ROW_LEN)
    out = _sc_permute_copy(src2d)
    return out.reshape(2, _BS, 1, _ROW_LEN)


# speedup vs baseline: 1.0239x; 1.0017x over previous
import jax
import jax.numpy as jnp
from jax.experimental import pallas as pl
from jax.experimental.pallas import tpu as pltpu
from jax.experimental.pallas import tpu_sc as plsc

_ROWS = 128
_ROW_LEN = 160000
_BS = _ROWS // 2
_NUM_WORKERS = 32
_RPW = _ROWS // _NUM_WORKERS
_NBUF = 8
_CHUNK = 8000
_CPR = _ROW_LEN // _CHUNK
_ITEMS = _RPW * _CPR

_PERM = (22, 18, 6, 26, 21, 45, 60, 39, 61, 49, 38, 27, 32, 57, 10, 63,
         35, 20, 24, 56, 52, 40, 51, 42, 55, 4, 31, 14, 0, 43, 34, 3,
         50, 5, 17, 37, 28, 2, 41, 23, 58, 44, 54, 48, 46, 36, 1, 8,
         16, 33, 30, 7, 19, 15, 9, 62, 13, 11, 59, 47, 25, 53, 12, 29)
_SRC_ROWS = _PERM + tuple(range(_BS, _ROWS))


def _sc_permute_copy(src2d):
    mesh = plsc.VectorSubcoreMesh(core_axis_name="c", subcore_axis_name="s")

    @pl.kernel(
        out_type=jax.ShapeDtypeStruct((_ROWS, _ROW_LEN), jnp.float32),
        mesh=mesh,
        compiler_params=pltpu.CompilerParams(use_tc_tiling_on_sc=False),
        scratch_types=[
            pltpu.VMEM((_NBUF, _CHUNK), jnp.float32),
            pltpu.SMEM((_RPW,), jnp.int32),
        ]
        + [pltpu.SemaphoreType.DMA] * (2 * _NBUF),
    )
    def k(src_hbm, out_hbm, bufs, srows, *sems):
        in_sems = sems[:_NBUF]
        out_sems = sems[_NBUF:]
        wid = jax.lax.axis_index("s") * 2 + jax.lax.axis_index("c")
        dst_base = wid * _RPW

        for w in range(_NUM_WORKERS):

            @pl.when(wid == w)
            def _(w=w):
                for i in range(_RPW):
                    srows[i] = _SRC_ROWS[w * _RPW + i]

        def in_copy(b, it):
            r = jax.lax.div(it, _CPR)
            off = jax.lax.mul(jax.lax.rem(it, _CPR), _CHUNK)
            return pltpu.make_async_copy(
                src_hbm.at[srows[r], pl.ds(off, _CHUNK)],
                bufs.at[b],
                in_sems[b],
            )

        def out_copy(b, it):
            r = jax.lax.div(it, _CPR)
            off = jax.lax.mul(jax.lax.rem(it, _CPR), _CHUNK)
            return pltpu.make_async_copy(
                bufs.at[b],
                out_hbm.at[dst_base + r, pl.ds(off, _CHUNK)],
                out_sems[b],
            )

        for b in range(_NBUF):
            in_copy(b, jnp.int32(b)).start()

        @pl.loop(0, _ITEMS, step=_NBUF)
        def _(t):
            for b in range(_NBUF):
                it = t + b
                in_copy(b, it).wait()
                out_copy(b, it).start()
            for b in range(_NBUF):
                it = t + b
                nxt = it + _NBUF

                @pl.when(nxt < _ITEMS)
                def _(b=b, it=it, nxt=nxt):
                    out_copy(b, it).wait()
                    in_copy(b, nxt).start()

        for b in range(_NBUF):
            out_copy(b, jnp.int32(_ITEMS - _NBUF + b)).wait()

    return k(src2d)


def kernel(sources):
    src2d = sources.reshape(_ROWS, _ROW_LEN)
    out = _sc_permute_copy(src2d)
    return out.reshape(2, _BS, 1, _ROW_LEN)


# speedup vs baseline: 1.0245x; 1.0006x over previous
import jax
import jax.numpy as jnp
from jax.experimental import pallas as pl
from jax.experimental.pallas import tpu as pltpu
from jax.experimental.pallas import tpu_sc as plsc

_ROWS = 128
_ROW_LEN = 160000
_BS = _ROWS // 2
_NUM_WORKERS = 32
_RPW = _ROWS // _NUM_WORKERS
_NBUF = 8
_CHUNK = 5000
_CPR = _ROW_LEN // _CHUNK
_ITEMS = _RPW * _CPR

_PERM = (22, 18, 6, 26, 21, 45, 60, 39, 61, 49, 38, 27, 32, 57, 10, 63,
         35, 20, 24, 56, 52, 40, 51, 42, 55, 4, 31, 14, 0, 43, 34, 3,
         50, 5, 17, 37, 28, 2, 41, 23, 58, 44, 54, 48, 46, 36, 1, 8,
         16, 33, 30, 7, 19, 15, 9, 62, 13, 11, 59, 47, 25, 53, 12, 29)
_SRC_ROWS = _PERM + tuple(range(_BS, _ROWS))


def _sc_permute_copy(src2d):
    mesh = plsc.VectorSubcoreMesh(core_axis_name="c", subcore_axis_name="s")

    @pl.kernel(
        out_type=jax.ShapeDtypeStruct((_ROWS, _ROW_LEN), jnp.float32),
        mesh=mesh,
        compiler_params=pltpu.CompilerParams(use_tc_tiling_on_sc=False),
        scratch_types=[
            pltpu.VMEM((_NBUF, _CHUNK), jnp.float32),
            pltpu.SMEM((_RPW,), jnp.int32),
        ]
        + [pltpu.SemaphoreType.DMA] * (2 * _NBUF),
    )
    def k(src_hbm, out_hbm, bufs, srows, *sems):
        in_sems = sems[:_NBUF]
        out_sems = sems[_NBUF:]
        wid = jax.lax.axis_index("s") * 2 + jax.lax.axis_index("c")
        dst_base = wid * _RPW

        for w in range(_NUM_WORKERS):

            @pl.when(wid == w)
            def _(w=w):
                for i in range(_RPW):
                    srows[i] = _SRC_ROWS[w * _RPW + i]

        def in_copy(b, it):
            r = jax.lax.div(it, _CPR)
            off = jax.lax.mul(jax.lax.rem(it, _CPR), _CHUNK)
            return pltpu.make_async_copy(
                src_hbm.at[srows[r], pl.ds(off, _CHUNK)],
                bufs.at[b],
                in_sems[b],
            )

        def out_copy(b, it):
            r = jax.lax.div(it, _CPR)
            off = jax.lax.mul(jax.lax.rem(it, _CPR), _CHUNK)
            return pltpu.make_async_copy(
                bufs.at[b],
                out_hbm.at[dst_base + r, pl.ds(off, _CHUNK)],
                out_sems[b],
            )

        for b in range(_NBUF):
            in_copy(b, jnp.int32(b)).start()

        @pl.loop(0, _ITEMS, step=_NBUF)
        def _(t):
            for b in range(_NBUF):
                it = t + b
                in_copy(b, it).wait()
                out_copy(b, it).start()
            for b in range(_NBUF):
                it = t + b
                nxt = it + _NBUF

                @pl.when(nxt < _ITEMS)
                def _(b=b, it=it, nxt=nxt):
                    out_copy(b, it).wait()
                    in_copy(b, nxt).start()

        for b in range(_NBUF):
            out_copy(b, jnp.int32(_ITEMS - _NBUF + b)).wait()

    return k(src2d)


def kernel(sources):
    src2d = sources.reshape(_ROWS, _ROW_LEN)
    out = _sc_permute_copy(src2d)
    return out.reshape(2, _BS, 1, _ROW_LEN)
